# Initial kernel scaffold; baseline (speedup 1.0000x reference)
import jax
import jax.numpy as jnp
from jax.experimental import pallas as pl


def kernel(x, edge_index, batch, target, gat_W, gat_att_src, gat_att_dst, gat_bias, gcn_W, gcn_bias, fcg1_W, fcg1_b, fcg2_W, fcg2_b, emb_table, conv_W, conv_b, fc1xt_W, fc1xt_b, fc1_W, fc1_b, fc2_W, fc2_b, fc3_W, fc3_b, fc4_W, fc4_b, out_W, out_b):
    raise NotImplementedError("write your pallas kernel here")


# speedup vs baseline: 1.0298x; 1.0298x over previous
import jax
import jax.numpy as jnp
from jax.experimental import pallas as pl
from jax.experimental.pallas import tpu as pltpu

N_NODES = 10000
N_GRAPHS = 128
HEADS = 10
FD = 78
NFILT = 32


def _head_body(xc_ref, w1, b1, w2, b2, w3, b3, w4, b4, wo, bo, out_ref):
    h = jnp.maximum(xc_ref[...] @ w1[...] + b1[...], 0.0)
    h = jnp.maximum(h @ w2[...] + b2[...], 0.0)
    h = jnp.maximum(h @ w3[...] + b3[...], 0.0)
    h = jnp.maximum(h @ w4[...] + b4[...], 0.0)
    out_ref[...] = h @ wo[...] + bo[...]


def _mlp_head(xc, fc1_W, fc1_b, fc2_W, fc2_b, fc3_W, fc3_b, fc4_W, fc4_b, out_W, out_b):
    return pl.pallas_call(
        _head_body,
        out_shape=jax.ShapeDtypeStruct((xc.shape[0], 1), jnp.float32),
    )(xc, fc1_W, fc1_b[None], fc2_W, fc2_b[None], fc3_W, fc3_b[None],
      fc4_W, fc4_b[None], out_W, out_b[None])


def _gat(x, src, dst, W, att_src, att_dst, bias):
    N = x.shape[0]
    h = (x @ W).reshape(N, HEADS, FD)
    a_src = jnp.sum(h * att_src[None], axis=-1)
    a_dst = jnp.sum(h * att_dst[None], axis=-1)
    e = a_src[src] + a_dst[dst]
    e = jax.nn.leaky_relu(e, 0.2)
    ex = jnp.exp(e)
    den = jax.ops.segment_sum(ex, dst, num_segments=N)
    alpha = ex / (den[dst] + 1e-16)
    msg = h[src] * alpha[:, :, None]
    out = jax.ops.segment_sum(msg, dst, num_segments=N)
    return out.reshape(N, HEADS * FD) + bias


def _gcn(x, src, dst, W, bias):
    N = x.shape[0]
    deg = jax.ops.segment_sum(jnp.ones(src.shape[0], dtype=jnp.float32), dst, num_segments=N)
    dinv = jnp.where(deg > 0, 1.0 / jnp.sqrt(deg), 0.0)
    norm = dinv[src] * dinv[dst]
    h = x @ W
    out = jax.ops.segment_sum(h[src] * norm[:, None], dst, num_segments=N)
    return out + bias


def kernel(x, edge_index, batch, target, gat_W, gat_att_src, gat_att_dst, gat_bias, gcn_W, gcn_bias, fcg1_W, fcg1_b, fcg2_W, fcg2_b, emb_table, conv_W, conv_b, fc1xt_W, fc1xt_b, fc1_W, fc1_b, fc2_W, fc2_b, fc3_W, fc3_b, fc4_W, fc4_b, out_W, out_b):
    N = x.shape[0]
    loop = jnp.arange(N, dtype=edge_index.dtype)
    src = jnp.concatenate([edge_index[0], loop])
    dst = jnp.concatenate([edge_index[1], loop])

    h = jax.nn.relu(_gat(x, src, dst, gat_W, gat_att_src, gat_att_dst, gat_bias))
    h = jax.nn.relu(_gcn(h, src, dst, gcn_W, gcn_bias))
    g = jax.ops.segment_sum(h, batch, num_segments=N_GRAPHS)
    g = jax.nn.relu(g @ fcg1_W + fcg1_b)
    g = g @ fcg2_W + fcg2_b

    emb = emb_table[target]
    xt = jnp.transpose(emb, (0, 2, 1))
    conv = jax.lax.conv_general_dilated(xt, conv_W, (1,), 'VALID',
                                        dimension_numbers=('NCH', 'OIH', 'NCH'))
    conv = conv + conv_b[None, :, None]
    conv = conv.reshape(-1, NFILT * 993)
    xt2 = conv @ fc1xt_W + fc1xt_b

    xc = jnp.concatenate([g, xt2], axis=1)
    return _mlp_head(xc, fc1_W, fc1_b, fc2_W, fc2_b, fc3_W, fc3_b, fc4_W, fc4_b, out_W, out_b)


# speedup vs baseline: 3.9102x; 3.7971x over previous
import functools

import jax
import jax.numpy as jnp
from jax import lax
from jax.experimental import pallas as pl
from jax.experimental.pallas import tpu as pltpu
from jax.experimental.pallas import tpu_sc as plsc

N_NODES = 10000
NP = 10240
N_GRAPHS = 128
HEADS = 10
FD = 78
FDP = 80
HF = 800
NFILT = 32
EP = 172032
E_REAL = 170000


def _k1_body(x_ref, w_ref, asrc_m_ref, adst_m_ref, h_ref, as_ref, ad_ref):
    h = x_ref[...] @ w_ref[...]
    h_ref[...] = h
    as_ref[...] = h @ asrc_m_ref[...]
    ad_ref[...] = h @ adst_m_ref[...]


def _gat_project(x_pad, w_pad, asrc_m, adst_m):
    blk = 2560
    grid = NP // blk
    return pl.pallas_call(
        _k1_body,
        grid=(grid,),
        in_specs=[
            pl.BlockSpec((blk, FD), lambda i: (i, 0)),
            pl.BlockSpec((FD, HF), lambda i: (0, 0)),
            pl.BlockSpec((HF, 16), lambda i: (0, 0)),
            pl.BlockSpec((HF, 16), lambda i: (0, 0)),
        ],
        out_specs=[
            pl.BlockSpec((blk, HF), lambda i: (i, 0)),
            pl.BlockSpec((blk, 16), lambda i: (i, 0)),
            pl.BlockSpec((blk, 16), lambda i: (i, 0)),
        ],
        out_shape=[
            jax.ShapeDtypeStruct((NP, HF), jnp.float32),
            jax.ShapeDtypeStruct((NP, 16), jnp.float32),
            jax.ShapeDtypeStruct((NP, 16), jnp.float32),
        ],
    )(x_pad, w_pad, asrc_m, adst_m)


def _sc_mesh():
    return plsc.VectorSubcoreMesh(core_axis_name="c", subcore_axis_name="s")


_G = 128
_TILE_E = EP // 32
_N_CH = _TILE_E // _G
_STRIPE = NP // 16


def _zero_rows(zbuf, ncols):
    for i in range(16):
        for v in range(ncols // 16):
            zbuf[i, pl.ds(v * 16, 16)] = jnp.zeros((16,), jnp.float32)


def _zero_stripe(zbuf, shared, base_r):
    @pl.loop(0, _STRIPE // 16)
    def _z(i):
        pltpu.sync_copy(zbuf, shared.at[pl.ds(base_r + i * 16, 16)])
    rem = _STRIPE % 16
    if rem:
        pltpu.sync_copy(zbuf.at[pl.ds(0, rem)],
                        shared.at[pl.ds(base_r + _STRIPE - rem, rem)])


def _k2_body(src_hbm, dst_hbm, as_hbm, ad_hbm, ex_hbm, denp_hbm,
             srcv, dstv, rows_s, rows_d, exb, zbuf, den_sh, sem):
    cid = lax.axis_index("c")
    sid = lax.axis_index("s")
    tile = cid * 16 + sid
    base_r = sid * _STRIPE

    _zero_rows(zbuf, 16)
    _zero_stripe(zbuf, den_sh, base_r)
    plsc.subcore_barrier()

    @pl.loop(0, _N_CH)
    def _chunk(j):
        base_e = tile * _TILE_E + j * _G
        pltpu.sync_copy(src_hbm.at[pl.ds(base_e, _G)], srcv)
        pltpu.sync_copy(dst_hbm.at[pl.ds(base_e, _G)], dstv)
        pltpu.async_copy(as_hbm.at[srcv], rows_s, sem).wait()
        pltpu.async_copy(ad_hbm.at[dstv], rows_d, sem).wait()

        @pl.loop(0, _G)
        def _edge(e):
            v = rows_s[e, :] + rows_d[e, :]
            v = jnp.where(v > 0.0, v, 0.2 * v)
            exb[e, :] = jnp.exp(v)

        pltpu.sync_copy(exb, ex_hbm.at[pl.ds(base_e, _G)])
        pltpu.sync_copy(exb, den_sh.at[dstv], add=True)

    plsc.subcore_barrier()
    pltpu.sync_copy(den_sh.at[pl.ds(base_r, _STRIPE)],
                    denp_hbm.at[cid].at[pl.ds(base_r, _STRIPE)])


def _edge_attention(src_p, dst_p, a_s, a_d):
    f = pl.kernel(
        _k2_body,
        out_type=[
            jax.ShapeDtypeStruct((EP, 16), jnp.float32),
            jax.ShapeDtypeStruct((2, NP, 16), jnp.float32),
        ],
        mesh=_sc_mesh(),
        scratch_types=[
            pltpu.VMEM((_G,), jnp.int32),
            pltpu.VMEM((_G,), jnp.int32),
            pltpu.VMEM((_G, 16), jnp.float32),
            pltpu.VMEM((_G, 16), jnp.float32),
            pltpu.VMEM((_G, 16), jnp.float32),
            pltpu.VMEM((16, 16), jnp.float32),
            pltpu.VMEM_SHARED((NP, 16), jnp.float32),
            pltpu.SemaphoreType.DMA,
        ],
        compiler_params=pltpu.CompilerParams(use_tc_tiling_on_sc=False),
    )
    return f(src_p, dst_p, a_s, a_d)


def _k3_body(denp_ref, recden_ref, dinv_ref):
    den = denp_ref[0] + denp_ref[1]
    recden_ref[...] = 1.0 / (den + 1e-16)
    deg = den[:, HEADS:HEADS + 1]
    dinv = jnp.where(deg > 0, lax.rsqrt(deg), 0.0)
    dinv_ref[...] = jnp.broadcast_to(dinv, dinv_ref.shape)


def _den_tables(den_p):
    blk = 2560
    return pl.pallas_call(
        _k3_body,
        grid=(NP // blk,),
        in_specs=[pl.BlockSpec((2, blk, 16), lambda i: (0, i, 0))],
        out_specs=[pl.BlockSpec((blk, 16), lambda i: (i, 0)),
                   pl.BlockSpec((blk, 16), lambda i: (i, 0))],
        out_shape=[jax.ShapeDtypeStruct((NP, 16), jnp.float32),
                   jax.ShapeDtypeStruct((NP, 16), jnp.float32)],
    )(den_p)


def _prep_body(src_hbm, dst_hbm, ex_hbm, recden_hbm, dinv_hbm,
               alpha_hbm, norm_hbm, srcv, dstv, exb, rdb, dsb, ddb, sem):
    cid = lax.axis_index("c")
    sid = lax.axis_index("s")
    tile = cid * 16 + sid

    @pl.loop(0, _N_CH)
    def _chunk(j):
        base_e = tile * _TILE_E + j * _G
        pltpu.sync_copy(src_hbm.at[pl.ds(base_e, _G)], srcv)
        pltpu.sync_copy(dst_hbm.at[pl.ds(base_e, _G)], dstv)
        pltpu.sync_copy(ex_hbm.at[pl.ds(base_e, _G)], exb)
        pltpu.async_copy(recden_hbm.at[dstv], rdb, sem).wait()
        pltpu.async_copy(dinv_hbm.at[srcv], dsb, sem).wait()
        pltpu.async_copy(dinv_hbm.at[dstv], ddb, sem).wait()

        @pl.loop(0, _G)
        def _edge(e):
            exb[e, :] = exb[e, :] * rdb[e, :]
            dsb[e, :] = dsb[e, :] * ddb[e, :]

        pltpu.sync_copy(exb, alpha_hbm.at[pl.ds(base_e, _G)])
        pltpu.sync_copy(dsb, norm_hbm.at[pl.ds(base_e, _G)])


def _edge_prep(src_p, dst_p, ex16, recden, dinv):
    f = pl.kernel(
        _prep_body,
        out_type=[jax.ShapeDtypeStruct((EP, 16), jnp.float32),
                  jax.ShapeDtypeStruct((EP, 16), jnp.float32)],
        mesh=_sc_mesh(),
        scratch_types=[
            pltpu.VMEM((_G,), jnp.int32),
            pltpu.VMEM((_G,), jnp.int32),
            pltpu.VMEM((_G, 16), jnp.float32),
            pltpu.VMEM((_G, 16), jnp.float32),
            pltpu.VMEM((_G, 16), jnp.float32),
            pltpu.VMEM((_G, 16), jnp.float32),
            pltpu.SemaphoreType.DMA,
        ],
        compiler_params=pltpu.CompilerParams(use_tc_tiling_on_sc=False),
    )
    return f(src_p, dst_p, ex16, recden, dinv)


def _msg_body(W, src_hbm, dst_hbm, coef_hbm, tab_hbm, out_hbm,
              srcv, dstv, coefb, hrows, zbuf, out_sh, sem):
    cid = lax.axis_index("c")
    sid = lax.axis_index("s")
    tile = cid * 16 + sid
    base_r = sid * _STRIPE

    _zero_rows(zbuf, W)
    _zero_stripe(zbuf, out_sh, base_r)
    plsc.subcore_barrier()

    @pl.loop(0, _N_CH)
    def _chunk(j):
        base_e = tile * _TILE_E + j * _G
        pltpu.sync_copy(src_hbm.at[pl.ds(base_e, _G)], srcv)
        pltpu.sync_copy(dst_hbm.at[pl.ds(base_e, _G)], dstv)
        pltpu.sync_copy(coef_hbm.at[pl.ds(base_e, _G)], coefb)
        pltpu.async_copy(tab_hbm.at[srcv], hrows, sem).wait()

        @pl.loop(0, _G)
        def _edge(e):
            nv = coefb[e, :]
            for v in range(W // 16):
                sl = pl.ds(v * 16, 16)
                hrows[e, sl] = hrows[e, sl] * nv

        pltpu.sync_copy(hrows, out_sh.at[dstv], add=True)

    plsc.subcore_barrier()
    pltpu.sync_copy(out_sh.at[pl.ds(base_r, _STRIPE)],
                    out_hbm.at[cid].at[pl.ds(base_r, _STRIPE)])


def _message_pass(W, src_p, dst_p, coef16, table):
    f = pl.kernel(
        functools.partial(_msg_body, W),
        out_type=jax.ShapeDtypeStruct((2, NP, W), jnp.float32),
        mesh=_sc_mesh(),
        scratch_types=[
            pltpu.VMEM((_G,), jnp.int32),
            pltpu.VMEM((_G,), jnp.int32),
            pltpu.VMEM((_G, 16), jnp.float32),
            pltpu.VMEM((_G, W), jnp.float32),
            pltpu.VMEM((16, W), jnp.float32),
            pltpu.VMEM_SHARED((NP, W), jnp.float32),
            pltpu.SemaphoreType.DMA,
        ],
        compiler_params=pltpu.CompilerParams(use_tc_tiling_on_sc=False),
    )
    return f(src_p, dst_p, coef16, table)


_PREC = jax.lax.Precision.HIGHEST


def _k36_body(a_ref, r_ref, out_ref):
    out_ref[...] = jnp.dot(a_ref[...], r_ref[...], precision=_PREC)


def _alpha_rep(alpha16):
    blk = 1344
    rep = jnp.zeros((16, 160), jnp.float32).at[
        jnp.repeat(jnp.arange(HEADS), 16), jnp.arange(160)].set(1.0)
    return pl.pallas_call(
        _k36_body,
        grid=(EP // blk,),
        in_specs=[pl.BlockSpec((blk, 16), lambda i: (i, 0)),
                  pl.BlockSpec((16, 160), lambda i: (0, 0))],
        out_specs=pl.BlockSpec((blk, 160), lambda i: (i, 0)),
        out_shape=jax.ShapeDtypeStruct((EP, 160), jnp.float32),
    )(alpha16, rep)


def _k5_body(gp_ref, bias_ref, w_ref, out_ref):
    acc = None
    for jp in range(HEADS):
        hj = gp_ref[jp, 0] + gp_ref[jp, 1] + bias_ref[0, pl.ds(jp * 80, 80)]
        hj = jnp.maximum(hj, 0.0)
        term = hj @ w_ref[0, pl.ds(jp * 80, 80), :]
        acc = term if acc is None else acc + term
    out_ref[0] = acc


def _gcn_matmul(gat10, bias_pad, w2p5):
    blk = 1024
    return pl.pallas_call(
        _k5_body,
        grid=(NP // blk, 5),
        in_specs=[
            pl.BlockSpec((HEADS, 2, blk, 80), lambda i, j: (0, 0, i, 0)),
            pl.BlockSpec((1, HF), lambda i, j: (0, 0)),
            pl.BlockSpec((1, HF, 160), lambda i, j: (j, 0, 0)),
        ],
        out_specs=pl.BlockSpec((1, blk, 160), lambda i, j: (j, i, 0)),
        out_shape=jax.ShapeDtypeStruct((5, NP, 160), jnp.float32),
    )(gat10, bias_pad, w2p5)


def _k7_body(op_ref, bias_ref, batch_ref, g_ref):
    i = pl.program_id(1)
    r = op_ref[0, 0] + op_ref[0, 1] + bias_ref[0]
    r = jnp.maximum(r, 0.0)
    b = batch_ref[0]
    blk = b.shape[1]
    oh = (lax.broadcasted_iota(jnp.int32, (N_GRAPHS, blk), 0)
          == jnp.broadcast_to(b, (N_GRAPHS, blk))).astype(jnp.float32)
    @pl.when(i == 0)
    def _():
        g_ref[...] = jnp.zeros_like(g_ref)
    g_ref[0] += jnp.dot(oh, r, precision=_PREC)


def _pool(out5, bias2_5, batch2d):
    blk = 2560
    return pl.pallas_call(
        _k7_body,
        grid=(5, NP // blk),
        in_specs=[
            pl.BlockSpec((1, 2, blk, 160), lambda j, i: (j, 0, i, 0)),
            pl.BlockSpec((1, 1, 160), lambda j, i: (j, 0, 0)),
            pl.BlockSpec((1, 1, blk), lambda j, i: (i, 0, 0)),
        ],
        out_specs=pl.BlockSpec((1, N_GRAPHS, 160), lambda j, i: (j, 0, 0)),
        out_shape=jax.ShapeDtypeStruct((5, N_GRAPHS, 160), jnp.float32),
    )(out5, bias2_5, batch2d)


def _k8_body(g_ref, w1_ref, b1_ref, w2_ref, b2_ref, out_ref):
    h = jnp.maximum(g_ref[...] @ w1_ref[...] + b1_ref[...], 0.0)
    out_ref[...] = h @ w2_ref[...] + b2_ref[...]


def _graph_head(g, fcg1p, fcg1_b, fcg2_W, fcg2_b):
    return pl.pallas_call(
        _k8_body,
        out_shape=jax.ShapeDtypeStruct((N_GRAPHS, 128), jnp.float32),
    )(g, fcg1p, fcg1_b[None], fcg2_W, fcg2_b[None])


def _k9_body(tgt_ref, embt_ref, w_ref, b_ref, out_ref):
    tg = tgt_ref[...]
    mask = (lax.broadcasted_iota(jnp.int32, (32, 1024), 1) < 993).astype(jnp.float32)
    for gg in range(8):
        t = tg[gg:gg + 1, :]
        oh = (lax.broadcasted_iota(jnp.int32, (32, 1024), 0)
              == jnp.broadcast_to(t, (32, 1024))).astype(jnp.float32)
        embt = jnp.dot(embt_ref[...], oh, precision=_PREC)
        acc = None
        for tt in range(8):
            y = jnp.dot(w_ref[tt], embt, precision=_PREC)
            y = pltpu.roll(y, (1024 - tt) % 1024, 1)
            acc = y if acc is None else acc + y
        out_ref[:, gg, :] = (acc + b_ref[...]) * mask


def _protein_conv(target_pad, embt_pad, wstack, conv_b):
    return pl.pallas_call(
        _k9_body,
        grid=(N_GRAPHS // 8,),
        in_specs=[
            pl.BlockSpec((8, 1024), lambda g: (g, 0)),
            pl.BlockSpec((128, 32), lambda g: (0, 0)),
            pl.BlockSpec((8, 32, 128), lambda g: (0, 0, 0)),
            pl.BlockSpec((32, 1), lambda g: (0, 0)),
        ],
        out_specs=pl.BlockSpec((32, 8, 1024), lambda g: (0, g, 0)),
        out_shape=jax.ShapeDtypeStruct((32, N_GRAPHS, 1024), jnp.float32),
    )(target_pad, embt_pad, wstack, conv_b)


def _k10_body(conv_ref, w_ref, out_ref):
    f = pl.program_id(0)
    @pl.when(f == 0)
    def _():
        out_ref[...] = jnp.zeros_like(out_ref)
    out_ref[...] += jnp.dot(conv_ref[0], w_ref[0], precision=_PREC)


def _protein_fc(conv, w1xp):
    return pl.pallas_call(
        _k10_body,
        grid=(NFILT,),
        in_specs=[
            pl.BlockSpec((1, N_GRAPHS, 1024), lambda f: (f, 0, 0)),
            pl.BlockSpec((1, 1024, 128), lambda f: (f, 0, 0)),
        ],
        out_specs=pl.BlockSpec((N_GRAPHS, 128), lambda f: (0, 0)),
        out_shape=jax.ShapeDtypeStruct((N_GRAPHS, 128), jnp.float32),
    )(conv, w1xp)


def _k11_body(g2_ref, xt2_ref, bxt_ref, w1a_ref, w1b_ref, b1_ref,
              w2_ref, b2_ref, w3_ref, b3_ref, w4_ref, b4_ref,
              wo_ref, bo_ref, out_ref):
    xt2 = xt2_ref[...] + bxt_ref[...]
    h = jnp.maximum(g2_ref[...] @ w1a_ref[...] + xt2 @ w1b_ref[...] + b1_ref[...], 0.0)
    h = jnp.maximum(h @ w2_ref[...] + b2_ref[...], 0.0)
    h = jnp.maximum(h @ w3_ref[...] + b3_ref[...], 0.0)
    h = jnp.maximum(h @ w4_ref[...] + b4_ref[...], 0.0)
    out_ref[...] = h @ wo_ref[...] + bo_ref[...]


def _final_head(g2, xt2, fc1xt_b, fc1_W, fc1_b, fc2_W, fc2_b, fc3_W, fc3_b,
                fc4_W, fc4_b, out_W, out_b):
    return pl.pallas_call(
        _k11_body,
        out_shape=jax.ShapeDtypeStruct((N_GRAPHS, 1), jnp.float32),
    )(g2, xt2, fc1xt_b[None], fc1_W[:128], fc1_W[128:], fc1_b[None],
      fc2_W, fc2_b[None], fc3_W, fc3_b[None], fc4_W, fc4_b[None],
      out_W, out_b[None])


def _prep_gat_weights(gat_W, att_src, att_dst):
    w = gat_W.reshape(FD, HEADS, FD)
    w_pad = jnp.pad(w, ((0, 0), (0, 0), (0, FDP - FD))).reshape(FD, HF)
    def head_mat(att):
        m = jnp.zeros((HEADS, FDP, 16), jnp.float32)
        m = m.at[jnp.arange(HEADS)[:, None], jnp.arange(FD)[None, :],
                 jnp.arange(HEADS)[:, None]].set(att)
        return m.reshape(HF, 16)
    return w_pad, head_mat(att_src), head_mat(att_dst)


def _pad_idx():
    r = jnp.arange(HEADS * FD)
    return (r // FD) * FDP + (r % FD)


def _pad_head_vec(v):
    return jnp.zeros((HF,), jnp.float32).at[_pad_idx()].set(v)


def _pad_gcn_w(gcn_W):
    w = jnp.zeros((HF, HF), jnp.float32)
    return w.at[_pad_idx(), :HEADS * FD].set(gcn_W)


def kernel(x, edge_index, batch, target, gat_W, gat_att_src, gat_att_dst, gat_bias, gcn_W, gcn_bias, fcg1_W, fcg1_b, fcg2_W, fcg2_b, emb_table, conv_W, conv_b, fc1xt_W, fc1xt_b, fc1_W, fc1_b, fc2_W, fc2_b, fc3_W, fc3_b, fc4_W, fc4_b, out_W, out_b):
    N = N_NODES
    x_pad = jnp.pad(x, ((0, NP - N), (0, 0)))
    w_pad, asrc_m, adst_m = _prep_gat_weights(gat_W, gat_att_src, gat_att_dst)
    h_pad, a_s, a_d = _gat_project(x_pad, w_pad, asrc_m, adst_m)

    loop = jnp.arange(N, dtype=edge_index.dtype)
    src = jnp.concatenate([edge_index[0], loop])
    dst = jnp.concatenate([edge_index[1], loop])

    src_p = jnp.full((EP,), N_NODES, jnp.int32).at[:E_REAL].set(src)
    dst_p = jnp.full((EP,), N_NODES, jnp.int32).at[:E_REAL].set(dst)
    ex16, den_p = _edge_attention(src_p, dst_p, a_s, a_d)
    recden, dinv = _den_tables(den_p)
    alpha16, norm16 = _edge_prep(src_p, dst_p, ex16, recden, dinv)

    alphab = _alpha_rep(alpha16)
    gat10 = jnp.stack([
        _message_pass(80, src_p, dst_p, alphab[:, 16 * k:16 * k + 16],
                      h_pad[:, 80 * k:80 * (k + 1)]) for k in range(HEADS)])
    bias_pad = _pad_head_vec(gat_bias)[None]
    w2p5 = jnp.transpose(_pad_gcn_w(gcn_W).reshape(HF, 5, 160), (1, 0, 2))
    h2_5 = _gcn_matmul(gat10, bias_pad, w2p5)

    out5 = jnp.stack([
        _message_pass(160, src_p, dst_p, norm16, h2_5[c]) for c in range(5)])
    bias2_5 = jnp.pad(gcn_bias, (0, HF - HEADS * FD)).reshape(5, 1, 160)
    batch2d = jnp.pad(batch, (0, NP - N), constant_values=200).reshape(4, 1, 2560)
    g5 = _pool(out5, bias2_5, batch2d)
    g = jnp.transpose(g5, (1, 0, 2)).reshape(N_GRAPHS, HF)

    g2 = _graph_head(g, jnp.pad(fcg1_W, ((0, HF - HEADS * FD), (0, 0))),
                     fcg1_b, fcg2_W, fcg2_b)

    target_pad = jnp.pad(target, ((0, 0), (0, 24)), constant_values=31)
    embt_pad = jnp.pad(emb_table.T, ((0, 0), (0, 6)))
    wstack = jnp.transpose(conv_W, (2, 0, 1))
    w1xp = jnp.pad(fc1xt_W.reshape(NFILT, 993, 128), ((0, 0), (0, 31), (0, 0)))
    conv = _protein_conv(target_pad, embt_pad, wstack, conv_b[:, None])
    xt2 = _protein_fc(conv, w1xp)

    return _final_head(g2, xt2, fc1xt_b, fc1_W, fc1_b, fc2_W, fc2_b,
                       fc3_W, fc3_b, fc4_W, fc4_b, out_W, out_b)
